# baseline (speedup 1.0000x reference)
import functools

import jax
import jax.numpy as jnp
from jax import lax
from jax.experimental import pallas as pl
from jax.experimental.pallas import tpu as pltpu
from jax.experimental.pallas import tpu_sc as plsc

NUM_CODES = 512
CODE_DIM = 32
TOK_BLK = 4096


def _split3(x):
    x1 = x.astype(jnp.bfloat16)
    r1 = x - x1.astype(jnp.float32)
    x2 = r1.astype(jnp.bfloat16)
    r2 = r1 - x2.astype(jnp.float32)
    x3 = r2.astype(jnp.bfloat16)
    return x1, x2, x3


def _argmin_kernel(z_ref, cb_ref, idx_ref):
    x = z_ref[:]
    c = cb_ref[:]
    cnorm2 = jnp.sum(c * c, axis=1, keepdims=True)
    h1, h2, h3 = _split3(-0.5 * cnorm2)
    x1, x2, x3 = _split3(x)
    c1, c2, c3 = _split3(c)
    one = jnp.ones((TOK_BLK, 1), jnp.bfloat16)
    z_cat = jnp.concatenate([x1, x1, x2, x1, x3, x2, one, one, one], axis=1)
    c_cat = jnp.concatenate([c1, c2, c1, c3, c1, c2, h1, h2, h3], axis=1)
    g = jax.lax.dot_general(
        c_cat, z_cat, (((1,), (1,)), ((), ())),
        preferred_element_type=jnp.float32)
    m = jnp.max(g, axis=0, keepdims=True)
    sub = jax.lax.broadcasted_iota(jnp.int32, (NUM_CODES, TOK_BLK), 0)
    idx_ref[0] = jnp.min(jnp.where(g == m, sub, NUM_CODES),
                         axis=0, keepdims=True)


def _make_gather(tok, D):
    info = plsc.get_sparse_core_info()
    NC, NS = info.num_cores, info.num_subcores
    NW = NC * NS
    b_per_w = tok // NW
    mesh = plsc.VectorSubcoreMesh(core_axis_name="c", subcore_axis_name="s")

    @functools.partial(
        pl.kernel, mesh=mesh,
        compiler_params=pltpu.CompilerParams(use_tc_tiling_on_sc=False),
        out_type=jax.ShapeDtypeStruct((tok, D), jnp.float32),
        scratch_types=[
            pltpu.VMEM((b_per_w,), jnp.int32),
            pltpu.VMEM((b_per_w, D), jnp.float32),
            pltpu.SemaphoreType.DMA,
        ],
    )
    def gather(table_hbm, idx_hbm, out_hbm, idx_v, rows_v, sem):
        wid = lax.axis_index("s") * NC + lax.axis_index("c")
        base = wid * b_per_w
        pltpu.sync_copy(idx_hbm.at[pl.ds(base, b_per_w)], idx_v)
        pltpu.async_copy(table_hbm.at[idx_v], rows_v, sem).wait()
        pltpu.sync_copy(rows_v, out_hbm.at[pl.ds(base, b_per_w)])

    return gather


def kernel(z_e, codebook):
    B, S, D = z_e.shape
    tok = B * S
    nblk = tok // TOK_BLK
    z2 = z_e.reshape(tok, D)
    idx = pl.pallas_call(
        _argmin_kernel,
        grid=(nblk,),
        in_specs=[
            pl.BlockSpec((TOK_BLK, D), lambda i: (i, 0)),
            pl.BlockSpec((NUM_CODES, D), lambda i: (0, 0)),
        ],
        out_specs=pl.BlockSpec((1, 1, TOK_BLK), lambda i: (i, 0, 0)),
        out_shape=jax.ShapeDtypeStruct((nblk, 1, TOK_BLK), jnp.int32),
    )(z2, codebook)
    idx_flat = idx.reshape(tok)
    zq = _make_gather(tok, D)(codebook, idx_flat)
    return zq.reshape(B, S, D), idx.reshape(B, S)


# speedup vs baseline: 5.1599x; 5.1599x over previous
import jax
import jax.numpy as jnp
from jax.experimental import pallas as pl

NUM_CODES = 512
CODE_DIM = 32


def _split3(x):
    x1 = x.astype(jnp.bfloat16)
    r1 = x - x1.astype(jnp.float32)
    x2 = r1.astype(jnp.bfloat16)
    r2 = r1 - x2.astype(jnp.float32)
    x3 = r2.astype(jnp.bfloat16)
    return x1, x2, x3


def _vq_kernel(zt_ref, ct_ref, zqt_ref, idx_ref):
    B, D, S = zt_ref.shape
    N = NUM_CODES
    ct = ct_ref[:]
    cnorm2 = jnp.sum(ct * ct, axis=0, keepdims=True)
    h1, h2, h3 = _split3(-0.5 * cnorm2)
    c1, c2, c3 = _split3(ct)
    c_cat = jnp.concatenate([c1, c2, c1, c3, c1, c2, h1, h2, h3], axis=0)
    c12 = jnp.concatenate([c1, c2], axis=0)
    sub = jax.lax.broadcasted_iota(jnp.int32, (N, S), 0)
    for b in range(B):
        x = zt_ref[b]
        x1, x2, x3 = _split3(x)
        one = jnp.ones((1, S), jnp.bfloat16)
        z_cat = jnp.concatenate([x1, x1, x2, x1, x3, x2, one, one, one],
                                axis=0)
        g = jax.lax.dot_general(
            c_cat, z_cat, (((0,), (0,)), ((), ())),
            preferred_element_type=jnp.float32)
        m = jnp.max(g, axis=0, keepdims=True)
        idxb = jnp.min(jnp.where(g == m, sub, N),
                       axis=0, keepdims=True)
        onehot = (sub == idxb).astype(jnp.bfloat16)
        zq2 = jax.lax.dot_general(
            c12, onehot, (((1,), (0,)), ((), ())),
            preferred_element_type=jnp.float32)
        zqt_ref[b] = zq2[:D] + zq2[D:]
        idx_ref[pl.ds(b, 1), :] = idxb


def kernel(z_e, codebook):
    B, S, D = z_e.shape
    zt = jnp.swapaxes(z_e, 1, 2)
    ct = codebook.T
    zqt, idx = pl.pallas_call(
        _vq_kernel,
        out_shape=[
            jax.ShapeDtypeStruct((B, D, S), jnp.float32),
            jax.ShapeDtypeStruct((B, S), jnp.int32),
        ],
    )(zt, ct)
    return jnp.swapaxes(zqt, 1, 2), idx
